# Initial kernel scaffold; baseline (speedup 1.0000x reference)
import jax
import jax.numpy as jnp
from jax.experimental import pallas as pl


def kernel(input_ids, weight):
    raise NotImplementedError("write your pallas kernel here")


# speedup vs baseline: 4.0897x; 4.0897x over previous
import functools

import jax
import jax.numpy as jnp
from jax import lax
from jax.experimental import pallas as pl
from jax.experimental.pallas import tpu as pltpu
from jax.experimental.pallas import tpu_sc as plsc

_NC = 2
_NS = 16
_NW = _NC * _NS
_CHUNK = 128


@functools.partial(jax.jit, static_argnames=("chunks_per_w", "d"))
def _sc_gather(weight, idx2d, *, chunks_per_w, d):
    total_rows = _NW * chunks_per_w * _CHUNK

    mesh = plsc.VectorSubcoreMesh(core_axis_name="c", subcore_axis_name="s")

    @functools.partial(
        pl.kernel,
        mesh=mesh,
        compiler_params=pltpu.CompilerParams(use_tc_tiling_on_sc=False),
        out_type=jax.ShapeDtypeStruct((total_rows, d), jnp.float32),
        scratch_types=[
            pltpu.VMEM((chunks_per_w, _CHUNK), jnp.int32),
            pltpu.VMEM((_CHUNK, d), jnp.float32),
            pltpu.SemaphoreType.DMA,
        ],
    )
    def run(w_hbm, idx_hbm, out_hbm, idx_v, rows_v, sem):
        wid = lax.axis_index("s") * _NC + lax.axis_index("c")
        pltpu.sync_copy(idx_hbm.at[wid], idx_v)
        row_base = wid * chunks_per_w * _CHUNK

        def chunk(j, carry):
            pltpu.async_copy(w_hbm.at[idx_v.at[j]], rows_v, sem).wait()
            pltpu.sync_copy(
                rows_v, out_hbm.at[pl.ds(row_base + j * _CHUNK, _CHUNK)]
            )
            return carry

        lax.fori_loop(0, chunks_per_w, chunk, 0)

    return run(weight, idx2d)


def kernel(input_ids, weight):
    out_shape = input_ids.shape + (weight.shape[1],)
    d = weight.shape[1]
    flat = input_ids.reshape(-1).astype(jnp.int32)
    b = flat.shape[0]
    tile = _NW * _CHUNK
    b_pad = ((b + tile - 1) // tile) * tile
    if b_pad != b:
        flat = jnp.pad(flat, (0, b_pad - b))
    chunks_per_w = b_pad // tile
    idx2d = flat.reshape(_NW, chunks_per_w, _CHUNK)
    out = _sc_gather(weight, idx2d, chunks_per_w=chunks_per_w, d=d)
    return out[:b].reshape(out_shape)


# speedup vs baseline: 4.6526x; 1.1376x over previous
import functools

import jax
import jax.numpy as jnp
from jax import lax
from jax.experimental import pallas as pl
from jax.experimental.pallas import tpu as pltpu
from jax.experimental.pallas import tpu_sc as plsc

_NC = 2
_NS = 16
_NW = _NC * _NS
_CHUNK = 128


@functools.partial(jax.jit, static_argnames=("chunks_per_w", "d"))
def _sc_gather(weight, idx2d, *, chunks_per_w, d):
    total_rows = _NW * chunks_per_w * _CHUNK

    mesh = plsc.VectorSubcoreMesh(core_axis_name="c", subcore_axis_name="s")

    if chunks_per_w % 5 == 0:
        nbuf = 5
    elif chunks_per_w % 2 == 0:
        nbuf = 2
    else:
        nbuf = 1
    groups = chunks_per_w // nbuf

    @functools.partial(
        pl.kernel,
        mesh=mesh,
        compiler_params=pltpu.CompilerParams(use_tc_tiling_on_sc=False),
        out_type=jax.ShapeDtypeStruct((total_rows, d), jnp.float32),
        scratch_types=[
            pltpu.VMEM((chunks_per_w, _CHUNK), jnp.int32),
            pltpu.VMEM((nbuf, _CHUNK, d), jnp.float32),
            pltpu.SemaphoreType.DMA((nbuf,)),
            pltpu.SemaphoreType.DMA((nbuf,)),
        ],
    )
    def run(w_hbm, idx_hbm, out_hbm, idx_v, rows_v, gsem, wsem):
        wid = lax.axis_index("s") * _NC + lax.axis_index("c")
        pltpu.sync_copy(idx_hbm.at[wid], idx_v)
        row_base = wid * chunks_per_w * _CHUNK

        def gather_start(b, j):
            pltpu.async_copy(w_hbm.at[idx_v.at[j]], rows_v.at[b], gsem.at[b])

        def gather_wait(b, j):
            pltpu.make_async_copy(
                w_hbm.at[idx_v.at[j]], rows_v.at[b], gsem.at[b]
            ).wait()

        def out_slice(j):
            return out_hbm.at[pl.ds(row_base + j * _CHUNK, _CHUNK)]

        def write_start(b, j):
            pltpu.async_copy(rows_v.at[b], out_slice(j), wsem.at[b])

        def write_wait(b, j):
            pltpu.make_async_copy(rows_v.at[b], out_slice(j), wsem.at[b]).wait()

        for b in range(nbuf):
            gather_start(b, b)

        def body(g, carry):
            for b in range(nbuf):
                j = g * nbuf + b
                gather_wait(b, j)
                write_start(b, j)
            for b in range(nbuf):
                j = g * nbuf + b
                write_wait(b, j)
                gather_start(b, j + nbuf)
            return carry

        lax.fori_loop(0, groups - 1, body, 0)

        g_last = groups - 1
        for b in range(nbuf):
            j = g_last * nbuf + b
            gather_wait(b, j)
            write_start(b, j)
        for b in range(nbuf):
            j = g_last * nbuf + b
            write_wait(b, j)

    return run(weight, idx2d)


def kernel(input_ids, weight):
    out_shape = input_ids.shape + (weight.shape[1],)
    d = weight.shape[1]
    flat = input_ids.reshape(-1).astype(jnp.int32)
    b = flat.shape[0]
    tile = _NW * _CHUNK
    b_pad = ((b + tile - 1) // tile) * tile
    if b_pad != b:
        flat = jnp.pad(flat, (0, b_pad - b))
    chunks_per_w = b_pad // tile
    idx2d = flat.reshape(_NW, chunks_per_w, _CHUNK)
    out = _sc_gather(weight, idx2d, chunks_per_w=chunks_per_w, d=d)
    return out[:b].reshape(out_shape)


# speedup vs baseline: 4.6625x; 1.0021x over previous
import functools

import jax
import jax.numpy as jnp
from jax import lax
from jax.experimental import pallas as pl
from jax.experimental.pallas import tpu as pltpu
from jax.experimental.pallas import tpu_sc as plsc

_NC = 2
_NS = 16
_NW = _NC * _NS
_CHUNK = 128


@functools.partial(jax.jit, static_argnames=("chunks_per_w", "d"))
def _sc_gather(weight, idx2d, *, chunks_per_w, d):
    total_rows = _NW * chunks_per_w * _CHUNK

    mesh = plsc.VectorSubcoreMesh(core_axis_name="c", subcore_axis_name="s")

    if chunks_per_w % 5 == 0:
        nbuf = 5
    elif chunks_per_w % 2 == 0:
        nbuf = 2
    else:
        nbuf = 1
    groups = chunks_per_w // nbuf
    n_chunks = _NW * chunks_per_w

    @functools.partial(
        pl.kernel,
        mesh=mesh,
        compiler_params=pltpu.CompilerParams(use_tc_tiling_on_sc=False),
        out_type=jax.ShapeDtypeStruct((n_chunks, _CHUNK, d), jnp.float32),
        scratch_types=[
            pltpu.VMEM((chunks_per_w, _CHUNK), jnp.int32),
            pltpu.VMEM((nbuf, _CHUNK, d), jnp.float32),
            pltpu.SemaphoreType.DMA((nbuf,)),
            pltpu.SemaphoreType.DMA((nbuf,)),
        ],
    )
    def run(w_hbm, idx_hbm, out_hbm, idx_v, rows_v, gsem, wsem):
        wid = lax.axis_index("s") * _NC + lax.axis_index("c")
        pltpu.sync_copy(idx_hbm.at[pl.ds(wid * chunks_per_w, chunks_per_w)], idx_v)
        chunk_base = wid * chunks_per_w

        def gather_start(b, j):
            pltpu.async_copy(w_hbm.at[idx_v.at[j]], rows_v.at[b], gsem.at[b])

        def gather_wait(b, j):
            pltpu.make_async_copy(
                w_hbm.at[idx_v.at[j]], rows_v.at[b], gsem.at[b]
            ).wait()

        def out_slice(j):
            return out_hbm.at[chunk_base + j]

        def wsrc(b):
            return rows_v.at[b]

        def write_start(b, j):
            pltpu.async_copy(wsrc(b), out_slice(j), wsem.at[b])

        def write_wait(b, j):
            pltpu.make_async_copy(wsrc(b), out_slice(j), wsem.at[b]).wait()

        for b in range(nbuf):
            gather_start(b, b)

        def body(g, carry):
            for b in range(nbuf):
                j = g * nbuf + b
                gather_wait(b, j)
                write_start(b, j)
            for b in range(nbuf):
                j = g * nbuf + b
                write_wait(b, j)
                gather_start(b, j + nbuf)
            return carry

        lax.fori_loop(0, groups - 1, body, 0)

        g_last = groups - 1
        for b in range(nbuf):
            j = g_last * nbuf + b
            gather_wait(b, j)
            write_start(b, j)
        for b in range(nbuf):
            j = g_last * nbuf + b
            write_wait(b, j)

    return run(weight, idx2d)


def kernel(input_ids, weight):
    out_shape = input_ids.shape + (weight.shape[1],)
    d = weight.shape[1]
    flat = input_ids.reshape(-1).astype(jnp.int32)
    b = flat.shape[0]
    tile = _NW * _CHUNK
    b_pad = ((b + tile - 1) // tile) * tile
    if b_pad != b:
        flat = jnp.pad(flat, (0, b_pad - b))
    chunks_per_w = b_pad // tile
    idx2d = flat.reshape(_NW * chunks_per_w, _CHUNK)
    out = _sc_gather(weight, idx2d, chunks_per_w=chunks_per_w, d=d)
    out = out.reshape(b_pad, d)
    return out[:b].reshape(out_shape)
